# Initial kernel scaffold; baseline (speedup 1.0000x reference)
import jax
import jax.numpy as jnp
from jax import lax
from jax.experimental import pallas as pl
from jax.experimental.pallas import tpu as pltpu
from jax.experimental.pallas import tpu_sc as plsc

_N = 10000
_E = 320000
_DIN = 128
_DH = 128
_DOUT = 64
_EPS = 1e-5

_NC = 2
_NS = 16
_NW = _NC * _NS
_EW = _E // _NW
_K = 80
_NCH = _EW // _K
_RS = _N // _NS

_mesh = plsc.VectorSubcoreMesh(core_axis_name="c", subcore_axis_name="s")


def _sc_agg_deg(y, src3, dst3, zD, z16, ones):
    D = _DH

    def body(y_hbm, src_hbm, dst_hbm, zD_hbm, z16_hbm, ones_hbm,
             part_out, deg_out, acc, dega, src_v, dst_v, rows_v, ones_v, sem):
        c = lax.axis_index("c")
        s = lax.axis_index("s")
        wid = s * _NC + c
        pltpu.sync_copy(zD_hbm, acc.at[pl.ds(s * _RS, _RS)])
        pltpu.sync_copy(z16_hbm, dega.at[pl.ds(s * _RS, _RS)])
        pltpu.sync_copy(ones_hbm, ones_v)
        pltpu.sync_copy(src_hbm.at[wid], src_v)
        pltpu.sync_copy(dst_hbm.at[wid], dst_v)
        plsc.subcore_barrier()

        def step(i, carry):
            pltpu.async_copy(y_hbm.at[src_v.at[i]], rows_v, sem).wait()
            pltpu.sync_copy(rows_v, acc.at[dst_v.at[i]], add=True)
            pltpu.sync_copy(ones_v, dega.at[dst_v.at[i]], add=True)
            return carry

        lax.fori_loop(0, _NCH, step, 0)
        plsc.subcore_barrier()
        pltpu.sync_copy(acc.at[pl.ds(s * _RS, _RS)],
                        part_out.at[c].at[pl.ds(s * _RS, _RS)])
        pltpu.sync_copy(dega.at[pl.ds(s * _RS, _RS)],
                        deg_out.at[c].at[pl.ds(s * _RS, _RS)])

    fn = pl.kernel(
        body,
        out_type=(jax.ShapeDtypeStruct((_NC, _N, D), jnp.float32),
                  jax.ShapeDtypeStruct((_NC, _N, 16), jnp.float32)),
        mesh=_mesh,
        scratch_types=(
            pltpu.VMEM_SHARED((_N, D), jnp.float32),
            pltpu.VMEM_SHARED((_N, 16), jnp.float32),
            pltpu.VMEM((_NCH, _K), jnp.int32),
            pltpu.VMEM((_NCH, _K), jnp.int32),
            pltpu.VMEM((_K, D), jnp.float32),
            pltpu.VMEM((_K, 16), jnp.float32),
            pltpu.SemaphoreType.DMA,
        ),
    )
    return fn(y, src3, dst3, zD, z16, ones)


def _sc_agg(y, src3, dst3, zD, D):
    def body(y_hbm, src_hbm, dst_hbm, zD_hbm, part_out,
             acc, src_v, dst_v, rows_v, sem):
        c = lax.axis_index("c")
        s = lax.axis_index("s")
        wid = s * _NC + c
        pltpu.sync_copy(zD_hbm, acc.at[pl.ds(s * _RS, _RS)])
        pltpu.sync_copy(src_hbm.at[wid], src_v)
        pltpu.sync_copy(dst_hbm.at[wid], dst_v)
        plsc.subcore_barrier()

        def step(i, carry):
            pltpu.async_copy(y_hbm.at[src_v.at[i]], rows_v, sem).wait()
            pltpu.sync_copy(rows_v, acc.at[dst_v.at[i]], add=True)
            return carry

        lax.fori_loop(0, _NCH, step, 0)
        plsc.subcore_barrier()
        pltpu.sync_copy(acc.at[pl.ds(s * _RS, _RS)],
                        part_out.at[c].at[pl.ds(s * _RS, _RS)])

    fn = pl.kernel(
        body,
        out_type=jax.ShapeDtypeStruct((_NC, _N, D), jnp.float32),
        mesh=_mesh,
        scratch_types=(
            pltpu.VMEM_SHARED((_N, D), jnp.float32),
            pltpu.VMEM((_NCH, _K), jnp.int32),
            pltpu.VMEM((_NCH, _K), jnp.int32),
            pltpu.VMEM((_K, D), jnp.float32),
            pltpu.SemaphoreType.DMA,
        ),
    )
    return fn(y, src3, dst3, zD)


_B = 1000


def _tc_first(x, wlT, wrT, bl2):
    def body(x_ref, wl_ref, wr_ref, bl_ref, y_ref, r_ref):
        xb = x_ref[...]
        y_ref[...] = jnp.dot(xb, wl_ref[...], preferred_element_type=jnp.float32)
        r_ref[...] = (jnp.dot(xb, wr_ref[...], preferred_element_type=jnp.float32)
                      + bl_ref[...])

    return pl.pallas_call(
        body,
        grid=(_N // _B,),
        in_specs=[
            pl.BlockSpec((_B, _DIN), lambda i: (i, 0)),
            pl.BlockSpec((_DIN, _DH), lambda i: (0, 0)),
            pl.BlockSpec((_DIN, _DH), lambda i: (0, 0)),
            pl.BlockSpec((1, _DH), lambda i: (0, 0)),
        ],
        out_specs=[
            pl.BlockSpec((_B, _DH), lambda i: (i, 0)),
            pl.BlockSpec((_B, _DH), lambda i: (i, 0)),
        ],
        out_shape=[
            jax.ShapeDtypeStruct((_N, _DH), jnp.float32),
            jax.ShapeDtypeStruct((_N, _DH), jnp.float32),
        ],
    )(x, wlT, wrT, bl2)


def _tc_combine(p, dp, r, wlT, wrT, bl2, g2, b2, Dn):
    D = _DH

    def body(p_ref, dp_ref, r_ref, wl_ref, wr_ref, bl_ref, g_ref, b_ref,
             y_ref, r2_ref):
        ssum = p_ref[0] + p_ref[1]
        deg = dp_ref[0, :, 0:1] + dp_ref[1, :, 0:1]
        inv = 1.0 / jnp.maximum(deg, 1.0)
        h = ssum * inv + r_ref[...]
        h = h * g_ref[...] + b_ref[...]
        h = jnp.maximum(h, 0.0)
        y_ref[...] = jnp.dot(h, wl_ref[...], preferred_element_type=jnp.float32)
        r2_ref[...] = (jnp.dot(h, wr_ref[...], preferred_element_type=jnp.float32)
                       + bl_ref[...])

    return pl.pallas_call(
        body,
        grid=(_N // _B,),
        in_specs=[
            pl.BlockSpec((_NC, _B, D), lambda i: (0, i, 0)),
            pl.BlockSpec((_NC, _B, 16), lambda i: (0, i, 0)),
            pl.BlockSpec((_B, D), lambda i: (i, 0)),
            pl.BlockSpec((D, Dn), lambda i: (0, 0)),
            pl.BlockSpec((D, Dn), lambda i: (0, 0)),
            pl.BlockSpec((1, Dn), lambda i: (0, 0)),
            pl.BlockSpec((1, D), lambda i: (0, 0)),
            pl.BlockSpec((1, D), lambda i: (0, 0)),
        ],
        out_specs=[
            pl.BlockSpec((_B, Dn), lambda i: (i, 0)),
            pl.BlockSpec((_B, Dn), lambda i: (i, 0)),
        ],
        out_shape=[
            jax.ShapeDtypeStruct((_N, Dn), jnp.float32),
            jax.ShapeDtypeStruct((_N, Dn), jnp.float32),
        ],
    )(p, dp, r, wlT, wrT, bl2, g2, b2)


def _tc_final(p, dp, r, g2, b2):
    D = _DOUT

    def body(p_ref, dp_ref, r_ref, g_ref, b_ref, o_ref):
        ssum = p_ref[0] + p_ref[1]
        deg = dp_ref[0, :, 0:1] + dp_ref[1, :, 0:1]
        inv = 1.0 / jnp.maximum(deg, 1.0)
        h = ssum * inv + r_ref[...]
        h = h * g_ref[...] + b_ref[...]
        nrm = jnp.sqrt(jnp.sum(h * h, axis=1, keepdims=True))
        o_ref[...] = h / jnp.maximum(nrm, 1e-12)

    return pl.pallas_call(
        body,
        grid=(_N // _B,),
        in_specs=[
            pl.BlockSpec((_NC, _B, D), lambda i: (0, i, 0)),
            pl.BlockSpec((_NC, _B, 16), lambda i: (0, i, 0)),
            pl.BlockSpec((_B, D), lambda i: (i, 0)),
            pl.BlockSpec((1, D), lambda i: (0, 0)),
            pl.BlockSpec((1, D), lambda i: (0, 0)),
        ],
        out_specs=pl.BlockSpec((_B, D), lambda i: (i, 0)),
        out_shape=jax.ShapeDtypeStruct((_N, D), jnp.float32),
    )(p, dp, r, g2, b2)


def kernel(x, edge_index, Wl1, bl1, Wr1, g1, b1, Wl2, bl2, Wr2, g2, b2,
           Wl3, bl3, Wr3, g3, b3):
    src3 = edge_index[0].reshape(_NW, _NCH, _K)
    dst3 = edge_index[1].reshape(_NW, _NCH, _K)
    zD = jnp.zeros((_RS, _DH), jnp.float32)
    z64 = jnp.zeros((_RS, _DOUT), jnp.float32)
    z16 = jnp.zeros((_RS, 16), jnp.float32)
    ones = jnp.ones((_K, 16), jnp.float32)
    bnscale = 1.0 / jnp.sqrt(jnp.float32(1.0 + _EPS))

    y1, r1 = _tc_first(x, Wl1.T, Wr1.T, bl1[None, :])
    p1, dp = _sc_agg_deg(y1, src3, dst3, zD, z16, ones)
    y2, r2 = _tc_combine(p1, dp, r1, Wl2.T, Wr2.T, bl2[None, :],
                         (g1 * bnscale)[None, :], b1[None, :], _DH)
    p2 = _sc_agg(y2, src3, dst3, zD, _DH)
    y3, r3 = _tc_combine(p2, dp, r2, Wl3.T, Wr3.T, bl3[None, :],
                         (g2 * bnscale)[None, :], b2[None, :], _DOUT)
    p3 = _sc_agg(y3, src3, dst3, z64, _DOUT)
    return _tc_final(p3, dp, r3, (g3 * bnscale)[None, :], b3[None, :])


# speedup vs baseline: 3.8737x; 3.8737x over previous
import jax
import jax.numpy as jnp
from jax import lax
from jax.experimental import pallas as pl
from jax.experimental.pallas import tpu as pltpu
from jax.experimental.pallas import tpu_sc as plsc

_N = 10000
_E = 320000
_DIN = 128
_DH = 128
_DOUT = 64
_EPS = 1e-5

_NC = 2
_NS = 16
_NW = _NC * _NS
_EW = _E // _NW
_K = 80
_NCH = _EW // _K
_NP = 10240
_RS = _NP // _NS

_mesh = plsc.VectorSubcoreMesh(core_axis_name="c", subcore_axis_name="s")


def _sc_agg_deg(y, src, dst, zD, z16, ones):
    D = _DH

    def body(y_hbm, src_hbm, dst_hbm, zD_hbm, z16_hbm, ones_hbm,
             part_out, deg_out, acc, dega, src_v, dst_v, rows_v, ones_v, sem):
        c = lax.axis_index("c")
        s = lax.axis_index("s")
        wid = s * _NC + c
        pltpu.sync_copy(zD_hbm, acc.at[pl.ds(s * _RS, _RS)])
        pltpu.sync_copy(z16_hbm, dega.at[pl.ds(s * _RS, _RS)])
        pltpu.sync_copy(ones_hbm, ones_v)
        plsc.subcore_barrier()
        base0 = wid * _EW

        def step(i, carry):
            base = base0 + i * _K
            pltpu.sync_copy(src_hbm.at[pl.ds(base, _K)], src_v)
            pltpu.sync_copy(dst_hbm.at[pl.ds(base, _K)], dst_v)
            pltpu.async_copy(y_hbm.at[src_v], rows_v, sem).wait()
            pltpu.sync_copy(rows_v, acc.at[dst_v], add=True)
            pltpu.sync_copy(ones_v, dega.at[dst_v], add=True)
            return carry

        lax.fori_loop(0, _NCH, step, 0)
        plsc.subcore_barrier()
        pltpu.sync_copy(acc.at[pl.ds(s * _RS, _RS)],
                        part_out.at[pl.ds(c * _NP + s * _RS, _RS)])
        pltpu.sync_copy(dega.at[pl.ds(s * _RS, _RS)],
                        deg_out.at[pl.ds(c * _NP + s * _RS, _RS)])

    fn = pl.kernel(
        body,
        out_type=(jax.ShapeDtypeStruct((_NC * _NP, D), jnp.float32),
                  jax.ShapeDtypeStruct((_NC * _NP, 16), jnp.float32)),
        mesh=_mesh,
        scratch_types=(
            pltpu.VMEM_SHARED((_NP, D), jnp.float32),
            pltpu.VMEM_SHARED((_NP, 16), jnp.float32),
            pltpu.VMEM((_K,), jnp.int32),
            pltpu.VMEM((_K,), jnp.int32),
            pltpu.VMEM((_K, D), jnp.float32),
            pltpu.VMEM((_K, 16), jnp.float32),
            pltpu.SemaphoreType.DMA,
        ),
    )
    p, d = fn(y, src, dst, zD, z16, ones)
    return p.reshape(_NC, _NP, D), d.reshape(_NC, _NP, 16)


def _sc_agg(y, src, dst, zD, D):
    def body(y_hbm, src_hbm, dst_hbm, zD_hbm, part_out,
             acc, src_v, dst_v, rows_v, sem):
        c = lax.axis_index("c")
        s = lax.axis_index("s")
        wid = s * _NC + c
        pltpu.sync_copy(zD_hbm, acc.at[pl.ds(s * _RS, _RS)])
        plsc.subcore_barrier()
        base0 = wid * _EW

        def step(i, carry):
            base = base0 + i * _K
            pltpu.sync_copy(src_hbm.at[pl.ds(base, _K)], src_v)
            pltpu.sync_copy(dst_hbm.at[pl.ds(base, _K)], dst_v)
            pltpu.async_copy(y_hbm.at[src_v], rows_v, sem).wait()
            pltpu.sync_copy(rows_v, acc.at[dst_v], add=True)
            return carry

        lax.fori_loop(0, _NCH, step, 0)
        plsc.subcore_barrier()
        pltpu.sync_copy(acc.at[pl.ds(s * _RS, _RS)],
                        part_out.at[pl.ds(c * _NP + s * _RS, _RS)])

    fn = pl.kernel(
        body,
        out_type=jax.ShapeDtypeStruct((_NC * _NP, D), jnp.float32),
        mesh=_mesh,
        scratch_types=(
            pltpu.VMEM_SHARED((_NP, D), jnp.float32),
            pltpu.VMEM((_K,), jnp.int32),
            pltpu.VMEM((_K,), jnp.int32),
            pltpu.VMEM((_K, D), jnp.float32),
            pltpu.SemaphoreType.DMA,
        ),
    )
    return fn(y, src, dst, zD).reshape(_NC, _NP, D)


_B = 1000


def _tc_first(x, wlT, wrT, bl2):
    def body(x_ref, wl_ref, wr_ref, bl_ref, y_ref, r_ref):
        xb = x_ref[...]
        y_ref[...] = jnp.dot(xb, wl_ref[...], preferred_element_type=jnp.float32)
        r_ref[...] = (jnp.dot(xb, wr_ref[...], preferred_element_type=jnp.float32)
                      + bl_ref[...])

    return pl.pallas_call(
        body,
        grid=(_N // _B,),
        in_specs=[
            pl.BlockSpec((_B, _DIN), lambda i: (i, 0)),
            pl.BlockSpec((_DIN, _DH), lambda i: (0, 0)),
            pl.BlockSpec((_DIN, _DH), lambda i: (0, 0)),
            pl.BlockSpec((1, _DH), lambda i: (0, 0)),
        ],
        out_specs=[
            pl.BlockSpec((_B, _DH), lambda i: (i, 0)),
            pl.BlockSpec((_B, _DH), lambda i: (i, 0)),
        ],
        out_shape=[
            jax.ShapeDtypeStruct((_N, _DH), jnp.float32),
            jax.ShapeDtypeStruct((_N, _DH), jnp.float32),
        ],
    )(x, wlT, wrT, bl2)


def _tc_combine(p, dp, r, wlT, wrT, bl2, g2, b2, Dn):
    D = _DH

    def body(p_ref, dp_ref, r_ref, wl_ref, wr_ref, bl_ref, g_ref, b_ref,
             y_ref, r2_ref):
        ssum = p_ref[0] + p_ref[1]
        deg = dp_ref[0, :, 0:1] + dp_ref[1, :, 0:1]
        inv = 1.0 / jnp.maximum(deg, 1.0)
        h = ssum * inv + r_ref[...]
        h = h * g_ref[...] + b_ref[...]
        h = jnp.maximum(h, 0.0)
        y_ref[...] = jnp.dot(h, wl_ref[...], preferred_element_type=jnp.float32)
        r2_ref[...] = (jnp.dot(h, wr_ref[...], preferred_element_type=jnp.float32)
                       + bl_ref[...])

    return pl.pallas_call(
        body,
        grid=(_N // _B,),
        in_specs=[
            pl.BlockSpec((_NC, _B, D), lambda i: (0, i, 0)),
            pl.BlockSpec((_NC, _B, 16), lambda i: (0, i, 0)),
            pl.BlockSpec((_B, D), lambda i: (i, 0)),
            pl.BlockSpec((D, Dn), lambda i: (0, 0)),
            pl.BlockSpec((D, Dn), lambda i: (0, 0)),
            pl.BlockSpec((1, Dn), lambda i: (0, 0)),
            pl.BlockSpec((1, D), lambda i: (0, 0)),
            pl.BlockSpec((1, D), lambda i: (0, 0)),
        ],
        out_specs=[
            pl.BlockSpec((_B, Dn), lambda i: (i, 0)),
            pl.BlockSpec((_B, Dn), lambda i: (i, 0)),
        ],
        out_shape=[
            jax.ShapeDtypeStruct((_N, Dn), jnp.float32),
            jax.ShapeDtypeStruct((_N, Dn), jnp.float32),
        ],
    )(p, dp, r, wlT, wrT, bl2, g2, b2)


def _tc_combine_h(p, dp, r, wrT, bl2, g2, b2):
    D = _DH

    def body(p_ref, dp_ref, r_ref, wr_ref, bl_ref, g_ref, b_ref,
             h_ref, r2_ref):
        ssum = p_ref[0] + p_ref[1]
        deg = dp_ref[0, :, 0:1] + dp_ref[1, :, 0:1]
        inv = 1.0 / jnp.maximum(deg, 1.0)
        h = ssum * inv + r_ref[...]
        h = h * g_ref[...] + b_ref[...]
        h = jnp.maximum(h, 0.0)
        h_ref[...] = h
        r2_ref[...] = (jnp.dot(h, wr_ref[...], preferred_element_type=jnp.float32)
                       + bl_ref[...])

    return pl.pallas_call(
        body,
        grid=(_N // _B,),
        in_specs=[
            pl.BlockSpec((_NC, _B, D), lambda i: (0, i, 0)),
            pl.BlockSpec((_NC, _B, 16), lambda i: (0, i, 0)),
            pl.BlockSpec((_B, D), lambda i: (i, 0)),
            pl.BlockSpec((D, _DOUT), lambda i: (0, 0)),
            pl.BlockSpec((1, _DOUT), lambda i: (0, 0)),
            pl.BlockSpec((1, D), lambda i: (0, 0)),
            pl.BlockSpec((1, D), lambda i: (0, 0)),
        ],
        out_specs=[
            pl.BlockSpec((_B, D), lambda i: (i, 0)),
            pl.BlockSpec((_B, _DOUT), lambda i: (i, 0)),
        ],
        out_shape=[
            jax.ShapeDtypeStruct((_N, D), jnp.float32),
            jax.ShapeDtypeStruct((_N, _DOUT), jnp.float32),
        ],
    )(p, dp, r, wrT, bl2, g2, b2)


def _tc_final(p, dp, r, wlT, g2, b2):
    D = _DOUT

    def body(p_ref, dp_ref, r_ref, wl_ref, g_ref, b_ref, o_ref):
        ssum = p_ref[0] + p_ref[1]
        deg = dp_ref[0, :, 0:1] + dp_ref[1, :, 0:1]
        inv = 1.0 / jnp.maximum(deg, 1.0)
        agg = ssum * inv
        h = (jnp.dot(agg, wl_ref[...], preferred_element_type=jnp.float32)
             + r_ref[...])
        h = h * g_ref[...] + b_ref[...]
        nrm = jnp.sqrt(jnp.sum(h * h, axis=1, keepdims=True))
        o_ref[...] = h / jnp.maximum(nrm, 1e-12)

    return pl.pallas_call(
        body,
        grid=(_N // _B,),
        in_specs=[
            pl.BlockSpec((_NC, _B, _DH), lambda i: (0, i, 0)),
            pl.BlockSpec((_NC, _B, 16), lambda i: (0, i, 0)),
            pl.BlockSpec((_B, D), lambda i: (i, 0)),
            pl.BlockSpec((_DH, D), lambda i: (0, 0)),
            pl.BlockSpec((1, D), lambda i: (0, 0)),
            pl.BlockSpec((1, D), lambda i: (0, 0)),
        ],
        out_specs=pl.BlockSpec((_B, D), lambda i: (i, 0)),
        out_shape=jax.ShapeDtypeStruct((_N, D), jnp.float32),
    )(p, dp, r, wlT, g2, b2)


def kernel(x, edge_index, Wl1, bl1, Wr1, g1, b1, Wl2, bl2, Wr2, g2, b2,
           Wl3, bl3, Wr3, g3, b3):
    src3 = edge_index[0]
    dst3 = edge_index[1]
    zD = jnp.zeros((_RS, _DH), jnp.float32)
    z16 = jnp.zeros((_RS, 16), jnp.float32)
    ones = jnp.ones((_K, 16), jnp.float32)
    bnscale = 1.0 / jnp.sqrt(jnp.float32(1.0 + _EPS))

    y1, r1 = _tc_first(x, Wl1.T, Wr1.T, bl1[None, :])
    p1 = _sc_agg(y1, src3, dst3, zD, _DH)
    d1 = jax.ops.segment_sum(jnp.ones((_E,), jnp.float32), dst3, num_segments=_N)
    dp = jnp.stack([jnp.pad(jnp.tile(d1[:, None], (1, 16)), ((0, _NP - _N), (0, 0))),
                    jnp.zeros((_NP, 16), jnp.float32)])
    y2, r2 = _tc_combine(p1, dp, r1, Wl2.T, Wr2.T, bl2[None, :],
                         (g1 * bnscale)[None, :], b1[None, :], _DH)
    p2 = _sc_agg(y2, src3, dst3, zD, _DH)
    h2, r3 = _tc_combine_h(p2, dp, r2, Wr3.T, bl3[None, :],
                           (g2 * bnscale)[None, :], b2[None, :])
    p3 = _sc_agg(h2, src3, dst3, zD, _DH)
    return _tc_final(p3, dp, r3, Wl3.T, (g3 * bnscale)[None, :], b3[None, :])


# speedup vs baseline: 5.3272x; 1.3752x over previous
import jax
import jax.numpy as jnp
from jax import lax
from jax.experimental import pallas as pl
from jax.experimental.pallas import tpu as pltpu
from jax.experimental.pallas import tpu_sc as plsc

_N = 10000
_E = 320000
_DIN = 128
_DH = 128
_DOUT = 64
_EPS = 1e-5

_NC = 2
_NS = 16
_NW = _NC * _NS
_EW = _E // _NW
_K = 80
_NCH = _EW // _K
_NP = 10240
_RS = _NP // _NS

_mesh = plsc.VectorSubcoreMesh(core_axis_name="c", subcore_axis_name="s")


def _sc_agg_deg(y, src, dst, zD, z16, ones):
    D = _DH

    def body(y_hbm, src_hbm, dst_hbm, zD_hbm, z16_hbm, ones_hbm,
             part_out, deg_out, acc, dega, src_v, dst_v, rows_v, ones_v, sem):
        c = lax.axis_index("c")
        s = lax.axis_index("s")
        wid = s * _NC + c
        pltpu.sync_copy(zD_hbm, acc.at[pl.ds(s * _RS, _RS)])
        pltpu.sync_copy(z16_hbm, dega.at[pl.ds(s * _RS, _RS)])
        pltpu.sync_copy(ones_hbm, ones_v)
        plsc.subcore_barrier()
        base0 = wid * _EW

        def step(i, carry):
            base = base0 + i * _K
            pltpu.sync_copy(src_hbm.at[pl.ds(base, _K)], src_v)
            pltpu.sync_copy(dst_hbm.at[pl.ds(base, _K)], dst_v)
            pltpu.async_copy(y_hbm.at[src_v], rows_v, sem).wait()
            pltpu.sync_copy(rows_v, acc.at[dst_v], add=True)
            pltpu.sync_copy(ones_v, dega.at[dst_v], add=True)
            return carry

        lax.fori_loop(0, _NCH, step, 0)
        plsc.subcore_barrier()
        pltpu.sync_copy(acc.at[pl.ds(s * _RS, _RS)],
                        part_out.at[pl.ds(c * _NP + s * _RS, _RS)])
        pltpu.sync_copy(dega.at[pl.ds(s * _RS, _RS)],
                        deg_out.at[pl.ds(c * _NP + s * _RS, _RS)])

    fn = pl.kernel(
        body,
        out_type=(jax.ShapeDtypeStruct((_NC * _NP, D), jnp.float32),
                  jax.ShapeDtypeStruct((_NC * _NP, 16), jnp.float32)),
        mesh=_mesh,
        scratch_types=(
            pltpu.VMEM_SHARED((_NP, D), jnp.float32),
            pltpu.VMEM_SHARED((_NP, 16), jnp.float32),
            pltpu.VMEM((_K,), jnp.int32),
            pltpu.VMEM((_K,), jnp.int32),
            pltpu.VMEM((_K, D), jnp.float32),
            pltpu.VMEM((_K, 16), jnp.float32),
            pltpu.SemaphoreType.DMA,
        ),
    )
    p, d = fn(y, src, dst, zD, z16, ones)
    return p.reshape(_NC, _NP, D), d.reshape(_NC, _NP, 16)


def _sc_agg(y, src, dst, zD, D):
    def body(y_hbm, src_hbm, dst_hbm, zD_hbm, part_out,
             acc, src_v, dst_v, rows_v, gsem0, gsem1):
        c = lax.axis_index("c")
        s = lax.axis_index("s")
        wid = s * _NC + c
        pltpu.sync_copy(zD_hbm, acc.at[pl.ds(s * _RS, _RS)])
        plsc.subcore_barrier()
        base0 = wid * _EW
        gsem = (gsem0, gsem1)

        def load_and_gather(cc, b):
            base = base0 + cc * _K
            pltpu.sync_copy(src_hbm.at[pl.ds(base, _K)], src_v.at[b])
            pltpu.sync_copy(dst_hbm.at[pl.ds(base, _K)], dst_v.at[b])
            pltpu.async_copy(y_hbm.at[src_v.at[b]], rows_v.at[b], gsem[b])

        def drain(b):
            pltpu.make_async_copy(y_hbm.at[src_v.at[b]], rows_v.at[b],
                                  gsem[b]).wait()
            pltpu.sync_copy(rows_v.at[b], acc.at[dst_v.at[b]], add=True)

        load_and_gather(0, 0)

        @pl.loop(0, (_NCH - 1) // 2)
        def _pair(g):
            c1 = 1 + 2 * g
            load_and_gather(c1, 1)
            drain(0)
            load_and_gather(c1 + 1, 0)
            drain(1)

        drain(0)
        plsc.subcore_barrier()
        pltpu.sync_copy(acc.at[pl.ds(s * _RS, _RS)],
                        part_out.at[pl.ds(c * _NP + s * _RS, _RS)])

    fn = pl.kernel(
        body,
        out_type=jax.ShapeDtypeStruct((_NC * _NP, D), jnp.float32),
        mesh=_mesh,
        scratch_types=(
            pltpu.VMEM_SHARED((_NP, D), jnp.float32),
            pltpu.VMEM((2, _K), jnp.int32),
            pltpu.VMEM((2, _K), jnp.int32),
            pltpu.VMEM((2, _K, D), jnp.float32),
            pltpu.SemaphoreType.DMA,
            pltpu.SemaphoreType.DMA,
        ),
    )
    return fn(y, src, dst, zD).reshape(_NC, _NP, D)


_B = 1000


def _tc_first(x, wlT, wrT, bl2):
    def body(x_ref, wl_ref, wr_ref, bl_ref, y_ref, r_ref):
        xb = x_ref[...]
        y_ref[...] = jnp.dot(xb, wl_ref[...], preferred_element_type=jnp.float32)
        r_ref[...] = (jnp.dot(xb, wr_ref[...], preferred_element_type=jnp.float32)
                      + bl_ref[...])

    return pl.pallas_call(
        body,
        grid=(_N // _B,),
        in_specs=[
            pl.BlockSpec((_B, _DIN), lambda i: (i, 0)),
            pl.BlockSpec((_DIN, _DH), lambda i: (0, 0)),
            pl.BlockSpec((_DIN, _DH), lambda i: (0, 0)),
            pl.BlockSpec((1, _DH), lambda i: (0, 0)),
        ],
        out_specs=[
            pl.BlockSpec((_B, _DH), lambda i: (i, 0)),
            pl.BlockSpec((_B, _DH), lambda i: (i, 0)),
        ],
        out_shape=[
            jax.ShapeDtypeStruct((_N, _DH), jnp.float32),
            jax.ShapeDtypeStruct((_N, _DH), jnp.float32),
        ],
    )(x, wlT, wrT, bl2)


def _tc_combine(p, dp, r, wlT, wrT, bl2, g2, b2, Dn):
    D = _DH

    def body(p_ref, dp_ref, r_ref, wl_ref, wr_ref, bl_ref, g_ref, b_ref,
             y_ref, r2_ref):
        ssum = p_ref[0] + p_ref[1]
        deg = dp_ref[0, :, 0:1] + dp_ref[1, :, 0:1]
        inv = 1.0 / jnp.maximum(deg, 1.0)
        h = ssum * inv + r_ref[...]
        h = h * g_ref[...] + b_ref[...]
        h = jnp.maximum(h, 0.0)
        y_ref[...] = jnp.dot(h, wl_ref[...], preferred_element_type=jnp.float32)
        r2_ref[...] = (jnp.dot(h, wr_ref[...], preferred_element_type=jnp.float32)
                       + bl_ref[...])

    return pl.pallas_call(
        body,
        grid=(_N // _B,),
        in_specs=[
            pl.BlockSpec((_NC, _B, D), lambda i: (0, i, 0)),
            pl.BlockSpec((_NC, _B, 16), lambda i: (0, i, 0)),
            pl.BlockSpec((_B, D), lambda i: (i, 0)),
            pl.BlockSpec((D, Dn), lambda i: (0, 0)),
            pl.BlockSpec((D, Dn), lambda i: (0, 0)),
            pl.BlockSpec((1, Dn), lambda i: (0, 0)),
            pl.BlockSpec((1, D), lambda i: (0, 0)),
            pl.BlockSpec((1, D), lambda i: (0, 0)),
        ],
        out_specs=[
            pl.BlockSpec((_B, Dn), lambda i: (i, 0)),
            pl.BlockSpec((_B, Dn), lambda i: (i, 0)),
        ],
        out_shape=[
            jax.ShapeDtypeStruct((_N, Dn), jnp.float32),
            jax.ShapeDtypeStruct((_N, Dn), jnp.float32),
        ],
    )(p, dp, r, wlT, wrT, bl2, g2, b2)


def _tc_combine_h(p, dp, r, wrT, bl2, g2, b2):
    D = _DH

    def body(p_ref, dp_ref, r_ref, wr_ref, bl_ref, g_ref, b_ref,
             h_ref, r2_ref):
        ssum = p_ref[0] + p_ref[1]
        deg = dp_ref[0, :, 0:1] + dp_ref[1, :, 0:1]
        inv = 1.0 / jnp.maximum(deg, 1.0)
        h = ssum * inv + r_ref[...]
        h = h * g_ref[...] + b_ref[...]
        h = jnp.maximum(h, 0.0)
        h_ref[...] = h
        r2_ref[...] = (jnp.dot(h, wr_ref[...], preferred_element_type=jnp.float32)
                       + bl_ref[...])

    return pl.pallas_call(
        body,
        grid=(_N // _B,),
        in_specs=[
            pl.BlockSpec((_NC, _B, D), lambda i: (0, i, 0)),
            pl.BlockSpec((_NC, _B, 16), lambda i: (0, i, 0)),
            pl.BlockSpec((_B, D), lambda i: (i, 0)),
            pl.BlockSpec((D, _DOUT), lambda i: (0, 0)),
            pl.BlockSpec((1, _DOUT), lambda i: (0, 0)),
            pl.BlockSpec((1, D), lambda i: (0, 0)),
            pl.BlockSpec((1, D), lambda i: (0, 0)),
        ],
        out_specs=[
            pl.BlockSpec((_B, D), lambda i: (i, 0)),
            pl.BlockSpec((_B, _DOUT), lambda i: (i, 0)),
        ],
        out_shape=[
            jax.ShapeDtypeStruct((_N, D), jnp.float32),
            jax.ShapeDtypeStruct((_N, _DOUT), jnp.float32),
        ],
    )(p, dp, r, wrT, bl2, g2, b2)


def _tc_final(p, dp, r, wlT, g2, b2):
    D = _DOUT

    def body(p_ref, dp_ref, r_ref, wl_ref, g_ref, b_ref, o_ref):
        ssum = p_ref[0] + p_ref[1]
        deg = dp_ref[0, :, 0:1] + dp_ref[1, :, 0:1]
        inv = 1.0 / jnp.maximum(deg, 1.0)
        agg = ssum * inv
        h = (jnp.dot(agg, wl_ref[...], preferred_element_type=jnp.float32)
             + r_ref[...])
        h = h * g_ref[...] + b_ref[...]
        nrm = jnp.sqrt(jnp.sum(h * h, axis=1, keepdims=True))
        o_ref[...] = h / jnp.maximum(nrm, 1e-12)

    return pl.pallas_call(
        body,
        grid=(_N // _B,),
        in_specs=[
            pl.BlockSpec((_NC, _B, _DH), lambda i: (0, i, 0)),
            pl.BlockSpec((_NC, _B, 16), lambda i: (0, i, 0)),
            pl.BlockSpec((_B, D), lambda i: (i, 0)),
            pl.BlockSpec((_DH, D), lambda i: (0, 0)),
            pl.BlockSpec((1, D), lambda i: (0, 0)),
            pl.BlockSpec((1, D), lambda i: (0, 0)),
        ],
        out_specs=pl.BlockSpec((_B, D), lambda i: (i, 0)),
        out_shape=jax.ShapeDtypeStruct((_N, D), jnp.float32),
    )(p, dp, r, wlT, g2, b2)


def kernel(x, edge_index, Wl1, bl1, Wr1, g1, b1, Wl2, bl2, Wr2, g2, b2,
           Wl3, bl3, Wr3, g3, b3):
    src3 = edge_index[0]
    dst3 = edge_index[1]
    zD = jnp.zeros((_RS, _DH), jnp.float32)
    z16 = jnp.zeros((_RS, 16), jnp.float32)
    ones = jnp.ones((_K, 16), jnp.float32)
    bnscale = 1.0 / jnp.sqrt(jnp.float32(1.0 + _EPS))

    y1, r1 = _tc_first(x, Wl1.T, Wr1.T, bl1[None, :])
    p1 = _sc_agg(y1, src3, dst3, zD, _DH)
    d1 = jax.ops.segment_sum(jnp.ones((_E,), jnp.float32), dst3, num_segments=_N)
    dp = jnp.stack([jnp.pad(jnp.tile(d1[:, None], (1, 16)), ((0, _NP - _N), (0, 0))),
                    jnp.zeros((_NP, 16), jnp.float32)])
    y2, r2 = _tc_combine(p1, dp, r1, Wl2.T, Wr2.T, bl2[None, :],
                         (g1 * bnscale)[None, :], b1[None, :], _DH)
    p2 = _sc_agg(y2, src3, dst3, zD, _DH)
    h2, r3 = _tc_combine_h(p2, dp, r2, Wr3.T, bl3[None, :],
                           (g2 * bnscale)[None, :], b2[None, :])
    p3 = _sc_agg(h2, src3, dst3, zD, _DH)
    return _tc_final(p3, dp, r3, Wl3.T, (g3 * bnscale)[None, :], b3[None, :])


# speedup vs baseline: 6.9680x; 1.3080x over previous
import jax
import jax.numpy as jnp
from jax import lax
from jax.experimental import pallas as pl
from jax.experimental.pallas import tpu as pltpu
from jax.experimental.pallas import tpu_sc as plsc

_N = 10000
_E = 320000
_DIN = 128
_DH = 128
_DOUT = 64
_EPS = 1e-5

_NC = 2
_NS = 16
_NW = _NC * _NS
_EW = _E // _NW
_K = 80
_NCH = _EW // _K
_NP = 10240
_RS = _NP // _NS

_mesh = plsc.VectorSubcoreMesh(core_axis_name="c", subcore_axis_name="s")


def _sc_deg(dst, zD, onesD):
    D = _DH

    def body(dst_hbm, zD_hbm, ones_hbm, deg_out, dega, dst_v, ones_v):
        c = lax.axis_index("c")
        s = lax.axis_index("s")
        wid = s * _NC + c
        pltpu.sync_copy(zD_hbm, dega.at[pl.ds(s * _RS, _RS)])
        pltpu.sync_copy(ones_hbm, ones_v)
        plsc.subcore_barrier()
        base0 = wid * _EW

        @pl.loop(0, _NCH)
        def _step(i):
            base = base0 + i * _K
            pltpu.sync_copy(dst_hbm.at[pl.ds(base, _K)], dst_v)
            pltpu.sync_copy(ones_v, dega.at[dst_v], add=True)

        plsc.subcore_barrier()
        pltpu.sync_copy(dega.at[pl.ds(s * _RS, _RS)],
                        deg_out.at[pl.ds(c * _NP + s * _RS, _RS)])

    fn = pl.kernel(
        body,
        out_type=jax.ShapeDtypeStruct((_NC * _NP, D), jnp.float32),
        mesh=_mesh,
        scratch_types=(
            pltpu.VMEM_SHARED((_NP, D), jnp.float32),
            pltpu.VMEM((_K,), jnp.int32),
            pltpu.VMEM((_K, D), jnp.float32),
        ),
    )
    return fn(dst, zD, onesD).reshape(_NC, _NP, D)


def _sc_agg(y, src, dst, zD, D):
    def body(y_hbm, src_hbm, dst_hbm, zD_hbm, part_out,
             acc, src_v, dst_v, rows_v, gsem0, gsem1):
        c = lax.axis_index("c")
        s = lax.axis_index("s")
        wid = s * _NC + c
        pltpu.sync_copy(zD_hbm, acc.at[pl.ds(s * _RS, _RS)])
        plsc.subcore_barrier()
        base0 = wid * _EW
        gsem = (gsem0, gsem1)

        def load_and_gather(cc, b):
            base = base0 + cc * _K
            pltpu.sync_copy(src_hbm.at[pl.ds(base, _K)], src_v.at[b])
            pltpu.sync_copy(dst_hbm.at[pl.ds(base, _K)], dst_v.at[b])
            pltpu.async_copy(y_hbm.at[src_v.at[b]], rows_v.at[b], gsem[b])

        def drain(b):
            pltpu.make_async_copy(y_hbm.at[src_v.at[b]], rows_v.at[b],
                                  gsem[b]).wait()
            pltpu.sync_copy(rows_v.at[b], acc.at[dst_v.at[b]], add=True)

        load_and_gather(0, 0)

        @pl.loop(0, (_NCH - 1) // 2)
        def _pair(g):
            c1 = 1 + 2 * g
            load_and_gather(c1, 1)
            drain(0)
            load_and_gather(c1 + 1, 0)
            drain(1)

        drain(0)
        plsc.subcore_barrier()
        pltpu.sync_copy(acc.at[pl.ds(s * _RS, _RS)],
                        part_out.at[pl.ds(c * _NP + s * _RS, _RS)])

    fn = pl.kernel(
        body,
        out_type=jax.ShapeDtypeStruct((_NC * _NP, D), jnp.float32),
        mesh=_mesh,
        scratch_types=(
            pltpu.VMEM_SHARED((_NP, D), jnp.float32),
            pltpu.VMEM((2, _K), jnp.int32),
            pltpu.VMEM((2, _K), jnp.int32),
            pltpu.VMEM((2, _K, D), jnp.float32),
            pltpu.SemaphoreType.DMA,
            pltpu.SemaphoreType.DMA,
        ),
    )
    return fn(y, src, dst, zD).reshape(_NC, _NP, D)


_B = 1000


def _tc_first(x, wlT, wrT, bl2):
    def body(x_ref, wl_ref, wr_ref, bl_ref, y_ref, r_ref):
        xb = x_ref[...]
        y_ref[...] = jnp.dot(xb, wl_ref[...], preferred_element_type=jnp.float32)
        r_ref[...] = (jnp.dot(xb, wr_ref[...], preferred_element_type=jnp.float32)
                      + bl_ref[...])

    return pl.pallas_call(
        body,
        grid=(_N // _B,),
        in_specs=[
            pl.BlockSpec((_B, _DIN), lambda i: (i, 0)),
            pl.BlockSpec((_DIN, _DH), lambda i: (0, 0)),
            pl.BlockSpec((_DIN, _DH), lambda i: (0, 0)),
            pl.BlockSpec((1, _DH), lambda i: (0, 0)),
        ],
        out_specs=[
            pl.BlockSpec((_B, _DH), lambda i: (i, 0)),
            pl.BlockSpec((_B, _DH), lambda i: (i, 0)),
        ],
        out_shape=[
            jax.ShapeDtypeStruct((_N, _DH), jnp.float32),
            jax.ShapeDtypeStruct((_N, _DH), jnp.float32),
        ],
    )(x, wlT, wrT, bl2)


def _tc_combine(p, dp, r, wlT, wrT, bl2, g2, b2, Dn):
    D = _DH

    def body(p_ref, dp_ref, r_ref, wl_ref, wr_ref, bl_ref, g_ref, b_ref,
             y_ref, r2_ref):
        ssum = p_ref[0] + p_ref[1]
        deg = dp_ref[0, :, 0:1] + dp_ref[1, :, 0:1]
        inv = 1.0 / jnp.maximum(deg, 1.0)
        h = ssum * inv + r_ref[...]
        h = h * g_ref[...] + b_ref[...]
        h = jnp.maximum(h, 0.0)
        y_ref[...] = jnp.dot(h, wl_ref[...], preferred_element_type=jnp.float32)
        r2_ref[...] = (jnp.dot(h, wr_ref[...], preferred_element_type=jnp.float32)
                       + bl_ref[...])

    return pl.pallas_call(
        body,
        grid=(_N // _B,),
        in_specs=[
            pl.BlockSpec((_NC, _B, D), lambda i: (0, i, 0)),
            pl.BlockSpec((_NC, _B, _DH), lambda i: (0, i, 0)),
            pl.BlockSpec((_B, D), lambda i: (i, 0)),
            pl.BlockSpec((D, Dn), lambda i: (0, 0)),
            pl.BlockSpec((D, Dn), lambda i: (0, 0)),
            pl.BlockSpec((1, Dn), lambda i: (0, 0)),
            pl.BlockSpec((1, D), lambda i: (0, 0)),
            pl.BlockSpec((1, D), lambda i: (0, 0)),
        ],
        out_specs=[
            pl.BlockSpec((_B, Dn), lambda i: (i, 0)),
            pl.BlockSpec((_B, Dn), lambda i: (i, 0)),
        ],
        out_shape=[
            jax.ShapeDtypeStruct((_N, Dn), jnp.float32),
            jax.ShapeDtypeStruct((_N, Dn), jnp.float32),
        ],
    )(p, dp, r, wlT, wrT, bl2, g2, b2)


def _tc_combine_h(p, dp, r, wrT, bl2, g2, b2):
    D = _DH

    def body(p_ref, dp_ref, r_ref, wr_ref, bl_ref, g_ref, b_ref,
             h_ref, r2_ref):
        ssum = p_ref[0] + p_ref[1]
        deg = dp_ref[0, :, 0:1] + dp_ref[1, :, 0:1]
        inv = 1.0 / jnp.maximum(deg, 1.0)
        h = ssum * inv + r_ref[...]
        h = h * g_ref[...] + b_ref[...]
        h = jnp.maximum(h, 0.0)
        h_ref[...] = h
        r2_ref[...] = (jnp.dot(h, wr_ref[...], preferred_element_type=jnp.float32)
                       + bl_ref[...])

    return pl.pallas_call(
        body,
        grid=(_N // _B,),
        in_specs=[
            pl.BlockSpec((_NC, _B, D), lambda i: (0, i, 0)),
            pl.BlockSpec((_NC, _B, _DH), lambda i: (0, i, 0)),
            pl.BlockSpec((_B, D), lambda i: (i, 0)),
            pl.BlockSpec((D, _DOUT), lambda i: (0, 0)),
            pl.BlockSpec((1, _DOUT), lambda i: (0, 0)),
            pl.BlockSpec((1, D), lambda i: (0, 0)),
            pl.BlockSpec((1, D), lambda i: (0, 0)),
        ],
        out_specs=[
            pl.BlockSpec((_B, D), lambda i: (i, 0)),
            pl.BlockSpec((_B, _DOUT), lambda i: (i, 0)),
        ],
        out_shape=[
            jax.ShapeDtypeStruct((_N, D), jnp.float32),
            jax.ShapeDtypeStruct((_N, _DOUT), jnp.float32),
        ],
    )(p, dp, r, wrT, bl2, g2, b2)


def _tc_final(p, dp, r, wlT, g2, b2):
    D = _DOUT

    def body(p_ref, dp_ref, r_ref, wl_ref, g_ref, b_ref, o_ref):
        ssum = p_ref[0] + p_ref[1]
        deg = dp_ref[0, :, 0:1] + dp_ref[1, :, 0:1]
        inv = 1.0 / jnp.maximum(deg, 1.0)
        agg = ssum * inv
        h = (jnp.dot(agg, wl_ref[...], preferred_element_type=jnp.float32)
             + r_ref[...])
        h = h * g_ref[...] + b_ref[...]
        nrm = jnp.sqrt(jnp.sum(h * h, axis=1, keepdims=True))
        o_ref[...] = h / jnp.maximum(nrm, 1e-12)

    return pl.pallas_call(
        body,
        grid=(_N // _B,),
        in_specs=[
            pl.BlockSpec((_NC, _B, _DH), lambda i: (0, i, 0)),
            pl.BlockSpec((_NC, _B, _DH), lambda i: (0, i, 0)),
            pl.BlockSpec((_B, D), lambda i: (i, 0)),
            pl.BlockSpec((_DH, D), lambda i: (0, 0)),
            pl.BlockSpec((1, D), lambda i: (0, 0)),
            pl.BlockSpec((1, D), lambda i: (0, 0)),
        ],
        out_specs=pl.BlockSpec((_B, D), lambda i: (i, 0)),
        out_shape=jax.ShapeDtypeStruct((_N, D), jnp.float32),
    )(p, dp, r, wlT, g2, b2)


def kernel(x, edge_index, Wl1, bl1, Wr1, g1, b1, Wl2, bl2, Wr2, g2, b2,
           Wl3, bl3, Wr3, g3, b3):
    src3 = edge_index[0]
    dst3 = edge_index[1]
    zD = jnp.zeros((_RS, _DH), jnp.float32)
    z16 = jnp.zeros((_RS, 16), jnp.float32)
    onesD = jnp.ones((_K, _DH), jnp.float32)
    bnscale = 1.0 / jnp.sqrt(jnp.float32(1.0 + _EPS))

    y1, r1 = _tc_first(x, Wl1.T, Wr1.T, bl1[None, :])
    p1 = _sc_agg(y1, src3, dst3, zD, _DH)
    dp = _sc_deg(dst3, zD, onesD)
    y2, r2 = _tc_combine(p1, dp, r1, Wl2.T, Wr2.T, bl2[None, :],
                         (g1 * bnscale)[None, :], b1[None, :], _DH)
    p2 = _sc_agg(y2, src3, dst3, zD, _DH)
    h2, r3 = _tc_combine_h(p2, dp, r2, Wr3.T, bl3[None, :],
                           (g2 * bnscale)[None, :], b2[None, :])
    p3 = _sc_agg(h2, src3, dst3, zD, _DH)
    return _tc_final(p3, dp, r3, Wl3.T, (g3 * bnscale)[None, :], b3[None, :])
